# Initial kernel scaffold; baseline (speedup 1.0000x reference)
import jax
import jax.numpy as jnp
from jax.experimental import pallas as pl


def kernel(candidate, word_table, rel_table):
    raise NotImplementedError("write your pallas kernel here")


# speedup vs baseline: 1.5582x; 1.5582x over previous
import functools

import jax
import jax.numpy as jnp
from jax import lax
from jax.experimental import pallas as pl
from jax.experimental.pallas import tpu as pltpu
from jax.experimental.pallas import tpu_sc as plsc

NUM_CORES = 2
NUM_SUBCORES = 16
NUM_WORKERS = NUM_CORES * NUM_SUBCORES
CHUNK = 128
EMBED = 64


@functools.cache
def _build(P):
    pairs_per_worker = P // NUM_WORKERS
    n_chunks = pairs_per_worker // CHUNK
    mesh = plsc.VectorSubcoreMesh(core_axis_name="c", subcore_axis_name="s")

    @functools.partial(
        pl.kernel,
        out_type=jax.ShapeDtypeStruct((P, 3 * EMBED), jnp.float32),
        mesh=mesh,
        scratch_types=[
            pltpu.VMEM((CHUNK,), jnp.int32),
            pltpu.VMEM((CHUNK,), jnp.int32),
            pltpu.VMEM((CHUNK,), jnp.int32),
            pltpu.VMEM((CHUNK, EMBED), jnp.float32),
            pltpu.VMEM((CHUNK, EMBED), jnp.float32),
            pltpu.VMEM((CHUNK, EMBED), jnp.float32),
            pltpu.SemaphoreType.DMA,
            pltpu.SemaphoreType.DMA,
            pltpu.SemaphoreType.DMA,
        ],
        compiler_params=pltpu.CompilerParams(use_tc_tiling_on_sc=False),
    )
    def gather_kernel(i0_hbm, i1_hbm, i2_hbm, word_hbm, rel_hbm, out_hbm,
                      i0_v, i1_v, i2_v, w0_v, w1_v, r_v, sem0, sem1, sem2):
        wid = lax.axis_index("s") * NUM_CORES + lax.axis_index("c")
        base = wid * pairs_per_worker

        @pl.loop(0, n_chunks)
        def _(t):
            off = base + t * CHUNK
            pltpu.sync_copy(i0_hbm.at[pl.ds(off, CHUNK)], i0_v)
            pltpu.sync_copy(i1_hbm.at[pl.ds(off, CHUNK)], i1_v)
            pltpu.sync_copy(i2_hbm.at[pl.ds(off, CHUNK)], i2_v)
            cp0 = pltpu.async_copy(word_hbm.at[i0_v], w0_v, sem0)
            cp1 = pltpu.async_copy(word_hbm.at[i1_v], w1_v, sem1)
            cp2 = pltpu.async_copy(rel_hbm.at[i2_v], r_v, sem2)
            cp0.wait()
            cp1.wait()
            cp2.wait()
            pltpu.sync_copy(w0_v, out_hbm.at[pl.ds(off, CHUNK), pl.ds(0, EMBED)])
            pltpu.sync_copy(w1_v, out_hbm.at[pl.ds(off, CHUNK), pl.ds(EMBED, EMBED)])
            pltpu.sync_copy(r_v, out_hbm.at[pl.ds(off, CHUNK), pl.ds(2 * EMBED, EMBED)])

    return gather_kernel


def kernel(candidate, word_table, rel_table):
    S, C, _ = candidate.shape
    P = S * C
    flat = candidate.reshape(P, 3)
    i0 = flat[:, 0]
    i1 = flat[:, 1]
    i2 = flat[:, 2]
    out = _build(P)(i0, i1, i2, word_table, rel_table)
    return out.reshape(S, C, 3 * EMBED)
